# Initial kernel scaffold; baseline (speedup 1.0000x reference)
import jax
import jax.numpy as jnp
from jax.experimental import pallas as pl


def kernel(a_, b_base_, b_diff_, t, b_prior_mean, b_prior_std_, indices, level_index):
    raise NotImplementedError("write your pallas kernel here")


# speedup vs baseline: 64.0870x; 64.0870x over previous
import functools
import math

import jax
import jax.numpy as jnp
from jax import lax
from jax.experimental import pallas as pl
from jax.experimental.pallas import tpu as pltpu
from jax.experimental.pallas import tpu_sc as plsc

N_ITEMS = 10000
N_PERSONS = 100000
N_GRADES = 5
N_RESP = 1000000
N_LEVELS = 10

_NC, _NS, _L = 2, 16, 16
_NW = _NC * _NS
_NPAD = 1 << 20
_W = _NPAD // _NW
_C = 2048
_NCHUNK = _W // _C
_G = 128

_LOG2PI = math.log(2.0 * math.pi)


def _softplus(x):
    return jnp.maximum(x, 0.0) + jnp.log(1.0 + jnp.exp(-jnp.abs(x)))


def _tables_body(a_ref, bb_ref, bd_ref, atab_ref, b6_ref):
    atab_ref[...] = _softplus(a_ref[...])
    sp = _softplus(bd_ref[...])
    c0 = bb_ref[...]
    c1 = c0 + sp[:, 0:1]
    c2 = c1 + sp[:, 1:2]
    c3 = c2 + sp[:, 2:3]
    lo = jnp.full_like(c0, -1000.0)
    hi = jnp.full_like(c0, 1000.0)
    b6_ref[...] = jnp.concatenate([lo, c0, c1, c2, c3, hi], axis=1)


def _make_tables(a2, bb, bd):
    return pl.pallas_call(
        _tables_body,
        out_shape=[
            jax.ShapeDtypeStruct((N_ITEMS, 1), jnp.float32),
            jax.ShapeDtypeStruct((N_ITEMS, 6), jnp.float32),
        ],
    )(a2, bb, bd)


def _sc_body(a_hbm, b6_hbm, t_hbm, item_hbm, person_hbm, resp_hbm, out_hbm,
             a_v, b6_v, item_v, person_v, resp_v, t_v, p_v, sem):
    wid = lax.axis_index("s") * _NC + lax.axis_index("c")
    base = wid * _W
    pltpu.sync_copy(a_hbm, a_v)
    pltpu.sync_copy(b6_hbm, b6_v)

    def chunk_body(g, carry):
        off = base + g * _C
        pltpu.sync_copy(item_hbm.at[pl.ds(off, _C)], item_v)
        pltpu.sync_copy(person_hbm.at[pl.ds(off, _C)], person_v)
        pltpu.sync_copy(resp_hbm.at[pl.ds(off, _C)], resp_v)
        cps = []
        for k in range(_C // _G):
            cps.append(pltpu.async_copy(
                t_hbm.at[person_v.at[pl.ds(k * _G, _G)]],
                t_v.at[pl.ds(k * _G, _G)], sem))
        for cp in cps:
            cp.wait()

        def vec_body(j, c2):
            s = pl.ds(j * _L, _L)
            it = item_v[s]
            rs = resp_v[s]
            fhi = it * 6 + rs
            a16 = plsc.load_gather(a_v, [it])
            bl = plsc.load_gather(b6_v, [fhi - 1])
            bu = plsc.load_gather(b6_v, [fhi])
            tt = t_v[s]
            zl = a16 * (tt - bl)
            zu = a16 * (tt - bu)
            sl = 1.0 / (1.0 + jnp.exp(-zl))
            su = 1.0 / (1.0 + jnp.exp(-zu))
            p_v[s] = sl - su
            return c2

        lax.fori_loop(0, _C // _L, vec_body, 0)
        pltpu.sync_copy(p_v, out_hbm.at[pl.ds(off, _C)])
        return carry

    lax.fori_loop(0, _NCHUNK, chunk_body, 0)


@functools.lru_cache(maxsize=1)
def _build_sc_gather():
    return pl.kernel(
        _sc_body,
        out_type=jax.ShapeDtypeStruct((_NPAD,), jnp.float32),
        mesh=plsc.VectorSubcoreMesh(
            core_axis_name="c", subcore_axis_name="s",
            num_cores=_NC, num_subcores=_NS),
        scratch_types=[
            pltpu.VMEM((N_ITEMS,), jnp.float32),
            pltpu.VMEM((6 * N_ITEMS,), jnp.float32),
            pltpu.VMEM((_C,), jnp.int32),
            pltpu.VMEM((_C,), jnp.int32),
            pltpu.VMEM((_C,), jnp.int32),
            pltpu.VMEM((_C,), jnp.float32),
            pltpu.VMEM((_C,), jnp.float32),
            pltpu.SemaphoreType.DMA,
        ],
        compiler_params=pltpu.CompilerParams(needs_layout_passes=False),
    )


def _finish_body(p_ref, a_ref, b6_ref, t_ref, bpm_ref, bps_ref, lvl_ref,
                 out_ref):
    rows, cols = p_ref.shape
    gid = (lax.broadcasted_iota(jnp.int32, (rows, cols), 0) * cols
           + lax.broadcasted_iota(jnp.int32, (rows, cols), 1))
    p = jnp.maximum(p_ref[...], 1e-37)
    ll = jnp.sum(jnp.where(gid < N_RESP, jnp.log(p), 0.0))

    a = a_ref[...]
    lp = jnp.sum(-0.5 * a * a) - 0.5 * _LOG2PI * N_ITEMS

    b = b6_ref[...][:, 1:5]
    lvl = lvl_ref[...]
    bpm = bpm_ref[...]
    bps = _softplus(bps_ref[...])
    mean = jnp.zeros((N_ITEMS, N_GRADES - 1), jnp.float32)
    std = jnp.zeros((N_ITEMS, N_GRADES - 1), jnp.float32)
    for l in range(N_LEVELS):
        m = (lvl == l).astype(jnp.float32)
        mean = mean + m * bpm[l:l + 1, :]
        std = std + m * bps[l:l + 1, :]
    z = (b - mean) / std
    lp += jnp.sum(-0.5 * z * z - jnp.log(std)) \
        - 0.5 * _LOG2PI * (N_ITEMS * (N_GRADES - 1))

    t = t_ref[...]
    lp += jnp.sum(-0.5 * t * t) - 0.5 * _LOG2PI * N_PERSONS
    lp += jnp.sum(-0.5 * bpm * bpm) - 0.5 * _LOG2PI * (N_LEVELS * (N_GRADES - 1))
    lp += jnp.sum(-2.0 * jnp.log(bps) - 1.0 / bps)

    out_ref[...] = jnp.reshape(-(ll + lp), (1, 1))


def _finish(p2d, atab, b6, t2d, bpm, bps_, lvl2d):
    return pl.pallas_call(
        _finish_body,
        out_shape=jax.ShapeDtypeStruct((1, 1), jnp.float32),
    )(p2d, atab, b6, t2d, bpm, bps_, lvl2d)


def kernel(a_, b_base_, b_diff_, t, b_prior_mean, b_prior_std_, indices,
           level_index):
    atab, b6 = _make_tables(a_.reshape(N_ITEMS, 1), b_base_, b_diff_)

    npad = _NPAD - N_RESP
    zpad = jnp.zeros((npad,), jnp.int32)
    item = jnp.concatenate([indices[:, 0].astype(jnp.int32), zpad])
    person = jnp.concatenate([indices[:, 1].astype(jnp.int32), zpad])
    resp = jnp.concatenate([indices[:, 2].astype(jnp.int32),
                            jnp.ones((npad,), jnp.int32)])

    p = _build_sc_gather()(atab.reshape(N_ITEMS), b6.reshape(6 * N_ITEMS),
                   t, item, person, resp)

    out = _finish(p.reshape(_NPAD // 128, 128), atab, b6,
                  t.reshape(800, 125), b_prior_mean, b_prior_std_,
                  level_index.reshape(N_ITEMS, 1).astype(jnp.int32))
    return out.reshape(())


# speedup vs baseline: 130.6752x; 2.0390x over previous
import functools
import math

import jax
import jax.numpy as jnp
from jax import lax
from jax.experimental import pallas as pl
from jax.experimental.pallas import tpu as pltpu
from jax.experimental.pallas import tpu_sc as plsc

N_ITEMS = 10000
N_PERSONS = 100000
N_GRADES = 5
N_RESP = 1000000
N_LEVELS = 10

_NC, _NS, _L = 2, 16, 16
_NW = _NC * _NS
_NPAD = 1 << 20
_W = _NPAD // _NW
_C = 4096
_NCHUNK = _W // _C

_LOG2PI = math.log(2.0 * math.pi)


def _softplus(x):
    return jnp.maximum(x, 0.0) + jnp.log(1.0 + jnp.exp(-jnp.abs(x)))


def _tables_body(a_ref, bb_ref, bd_ref, atab_ref, b6_ref):
    atab_ref[...] = _softplus(a_ref[...])
    sp = _softplus(bd_ref[...])
    c0 = bb_ref[...]
    c1 = c0 + sp[:, 0:1]
    c2 = c1 + sp[:, 1:2]
    c3 = c2 + sp[:, 2:3]
    lo = jnp.full_like(c0, -1000.0)
    hi = jnp.full_like(c0, 1000.0)
    b6_ref[...] = jnp.concatenate([lo, c0, c1, c2, c3, hi], axis=1)


def _make_tables(a2, bb, bd):
    return pl.pallas_call(
        _tables_body,
        out_shape=[
            jax.ShapeDtypeStruct((N_ITEMS, 1), jnp.float32),
            jax.ShapeDtypeStruct((N_ITEMS, 6), jnp.float32),
        ],
    )(a2, bb, bd)


def _sc_body(a_hbm, b6_hbm, t_hbm, item_hbm, person_hbm, resp_hbm, out_hbm,
             a_v, b6_v, t_sh,
             i0, i1, pn0, pn1, r0, r1, t0, t1, p0, p1,
             si0, si1, st0, st1, so0, so1):
    cid = lax.axis_index("c")
    sid = lax.axis_index("s")
    wid = sid * _NC + cid
    base = wid * _W

    @pl.when(sid == 0)
    def _():
        pltpu.sync_copy(t_hbm, t_sh)

    pltpu.sync_copy(a_hbm, a_v)
    pltpu.sync_copy(b6_hbm, b6_v)
    plsc.subcore_barrier()

    ib = (i0, i1)
    pb = (pn0, pn1)
    rb = (r0, r1)
    tb = (t0, t1)
    ob = (p0, p1)
    isem = (si0, si1)
    tsem = (st0, st1)
    osem = (so0, so1)
    descs = {}

    def fire_idx(g):
        off = base + g * _C
        b = g % 2
        descs[("i", g)] = [
            pltpu.async_copy(item_hbm.at[pl.ds(off, _C)], ib[b], isem[b]),
            pltpu.async_copy(person_hbm.at[pl.ds(off, _C)], pb[b], isem[b]),
            pltpu.async_copy(resp_hbm.at[pl.ds(off, _C)], rb[b], isem[b]),
        ]

    def fire_t(g):
        b = g % 2
        descs[("t", g)] = pltpu.async_copy(t_sh.at[pb[b]], tb[b], tsem[b])

    def compute(g):
        b = g % 2
        off = base + g * _C
        lane = lax.iota(jnp.int32, _L)

        def vec_body(j, carry):
            s = pl.ds(j * _L, _L)
            it = ib[b][s]
            rs = rb[b][s]
            fhi = it * 6 + rs
            a16 = plsc.load_gather(a_v, [it])
            bl = plsc.load_gather(b6_v, [fhi - 1])
            bu = plsc.load_gather(b6_v, [fhi])
            tt = tb[b][s]
            zl = jnp.maximum(a16 * (tt - bl), -30.0)
            zu = jnp.maximum(a16 * (tt - bu), -30.0)
            x = jnp.exp(-zl)
            y = jnp.exp(-zu)
            p = (y - x) / ((1.0 + x) * (1.0 + y))
            pos = off + j * _L + lane
            ob[b][s] = jnp.where(pos < N_RESP, p, 1.0)
            return carry

        lax.fori_loop(0, _C // _L, vec_body, 0)

    fire_idx(0)
    for d in descs[("i", 0)]:
        d.wait()
    fire_t(0)
    if _NCHUNK > 1:
        fire_idx(1)
    for g in range(_NCHUNK):
        descs[("t", g)].wait()
        if g + 1 < _NCHUNK:
            for d in descs[("i", g + 1)]:
                d.wait()
            fire_t(g + 1)
        if g >= 2:
            descs[("o", g - 2)].wait()
        compute(g)
        off = base + g * _C
        descs[("o", g)] = pltpu.async_copy(
            ob[g % 2], out_hbm.at[pl.ds(off, _C)], osem[g % 2])
        if g + 2 < _NCHUNK:
            fire_idx(g + 2)
    for g in (_NCHUNK - 2, _NCHUNK - 1):
        if g >= 0:
            descs[("o", g)].wait()


@functools.lru_cache(maxsize=1)
def _build_sc_gather():
    return pl.kernel(
        _sc_body,
        out_type=jax.ShapeDtypeStruct((_NPAD,), jnp.float32),
        mesh=plsc.VectorSubcoreMesh(
            core_axis_name="c", subcore_axis_name="s",
            num_cores=_NC, num_subcores=_NS),
        scratch_types=[
            pltpu.VMEM((N_ITEMS,), jnp.float32),
            pltpu.VMEM((6 * N_ITEMS,), jnp.float32),
            pltpu.VMEM_SHARED((N_PERSONS,), jnp.float32),
            pltpu.VMEM((_C,), jnp.int32),
            pltpu.VMEM((_C,), jnp.int32),
            pltpu.VMEM((_C,), jnp.int32),
            pltpu.VMEM((_C,), jnp.int32),
            pltpu.VMEM((_C,), jnp.int32),
            pltpu.VMEM((_C,), jnp.int32),
            pltpu.VMEM((_C,), jnp.float32),
            pltpu.VMEM((_C,), jnp.float32),
            pltpu.VMEM((_C,), jnp.float32),
            pltpu.VMEM((_C,), jnp.float32),
            pltpu.SemaphoreType.DMA,
            pltpu.SemaphoreType.DMA,
            pltpu.SemaphoreType.DMA,
            pltpu.SemaphoreType.DMA,
            pltpu.SemaphoreType.DMA,
            pltpu.SemaphoreType.DMA,
        ],
        compiler_params=pltpu.CompilerParams(needs_layout_passes=False),
    )


def _finish_body(p_ref, a_ref, b6_ref, t_ref, bpm_ref, bps_ref, lvl_ref,
                 out_ref):
    p = jnp.maximum(p_ref[...], 1e-37)
    ll = jnp.sum(jnp.log(p))

    a = a_ref[...]
    lp = jnp.sum(-0.5 * a * a) - 0.5 * _LOG2PI * N_ITEMS

    b = b6_ref[...][:, 1:5]
    lvl = lvl_ref[...]
    bpm = bpm_ref[...]
    bps = _softplus(bps_ref[...])
    mean = jnp.zeros((N_ITEMS, N_GRADES - 1), jnp.float32)
    std = jnp.zeros((N_ITEMS, N_GRADES - 1), jnp.float32)
    for l in range(N_LEVELS):
        m = (lvl == l).astype(jnp.float32)
        mean = mean + m * bpm[l:l + 1, :]
        std = std + m * bps[l:l + 1, :]
    z = (b - mean) / std
    lp += jnp.sum(-0.5 * z * z - jnp.log(std)) \
        - 0.5 * _LOG2PI * (N_ITEMS * (N_GRADES - 1))

    t = t_ref[...]
    lp += jnp.sum(-0.5 * t * t) - 0.5 * _LOG2PI * N_PERSONS
    lp += jnp.sum(-0.5 * bpm * bpm) - 0.5 * _LOG2PI * (N_LEVELS * (N_GRADES - 1))
    lp += jnp.sum(-2.0 * jnp.log(bps) - 1.0 / bps)

    out_ref[...] = jnp.reshape(-(ll + lp), (1, 1))


def _finish(p2d, atab, b6, t2d, bpm, bps_, lvl2d):
    return pl.pallas_call(
        _finish_body,
        out_shape=jax.ShapeDtypeStruct((1, 1), jnp.float32),
    )(p2d, atab, b6, t2d, bpm, bps_, lvl2d)


def kernel(a_, b_base_, b_diff_, t, b_prior_mean, b_prior_std_, indices,
           level_index):
    atab, b6 = _make_tables(a_.reshape(N_ITEMS, 1), b_base_, b_diff_)

    npad = _NPAD - N_RESP
    zpad = jnp.zeros((npad,), jnp.int32)
    item = jnp.concatenate([indices[:, 0].astype(jnp.int32), zpad])
    person = jnp.concatenate([indices[:, 1].astype(jnp.int32), zpad])
    resp = jnp.concatenate([indices[:, 2].astype(jnp.int32),
                            jnp.ones((npad,), jnp.int32)])

    p = _build_sc_gather()(atab.reshape(N_ITEMS), b6.reshape(6 * N_ITEMS),
                           t, item, person, resp)

    out = _finish(p.reshape(_NPAD // 128, 128), atab, b6,
                  t.reshape(800, 125), b_prior_mean, b_prior_std_,
                  level_index.reshape(N_ITEMS, 1).astype(jnp.int32))
    return out.reshape(())


# speedup vs baseline: 155.1980x; 1.1877x over previous
import functools
import math

import jax
import jax.numpy as jnp
from jax import lax
from jax.experimental import pallas as pl
from jax.experimental.pallas import tpu as pltpu
from jax.experimental.pallas import tpu_sc as plsc

N_ITEMS = 10000
N_PERSONS = 100000
N_GRADES = 5
N_RESP = 1000000
N_LEVELS = 10

_NC, _NS, _L = 2, 16, 16
_NW = _NC * _NS
_NPAD = 1 << 20
_W = _NPAD // _NW
_C = 4096
_NCHUNK = _W // _C

_LOG2PI = math.log(2.0 * math.pi)
_LN2 = math.log(2.0)


def _softplus(x):
    return jnp.maximum(x, 0.0) + jnp.log(1.0 + jnp.exp(-jnp.abs(x)))


def _tables_body(a_ref, bb_ref, bd_ref, atab_ref, b6_ref):
    atab_ref[...] = _softplus(a_ref[...])
    sp = _softplus(bd_ref[...])
    c0 = bb_ref[...]
    c1 = c0 + sp[:, 0:1]
    c2 = c1 + sp[:, 1:2]
    c3 = c2 + sp[:, 2:3]
    lo = jnp.full_like(c0, -1000.0)
    hi = jnp.full_like(c0, 1000.0)
    b6_ref[...] = jnp.concatenate([lo, c0, c1, c2, c3, hi], axis=1)


def _make_tables(a2, bb, bd):
    return pl.pallas_call(
        _tables_body,
        out_shape=[
            jax.ShapeDtypeStruct((N_ITEMS, 1), jnp.float32),
            jax.ShapeDtypeStruct((N_ITEMS, 6), jnp.float32),
        ],
    )(a2, bb, bd)


def _ln16(p):
    bits = plsc.bitcast(p, jnp.int32)
    e = lax.shift_right_logical(bits, 23) - 127
    m = plsc.bitcast((bits & 0x007FFFFF) | 0x3F800000, jnp.float32)
    s = (m - 1.0) / (m + 1.0)
    s2 = s * s
    atanh = s * (1.0 + s2 * (1.0 / 3.0 + s2 * (0.2 + s2 * (1.0 / 7.0))))
    return e.astype(jnp.float32) * _LN2 + 2.0 * atanh


def _sc_body(a_hbm, b6_hbm, t_hbm, item_hbm, person_hbm, resp_hbm, out_hbm,
             a_v, b6_v, t_sh, acc_v,
             i0, i1, pn0, pn1, r0, r1, t0, t1,
             si0, si1, st0, st1, so):
    cid = lax.axis_index("c")
    sid = lax.axis_index("s")
    wid = sid * _NC + cid
    base = wid * _W

    @pl.when(sid == 0)
    def _():
        pltpu.sync_copy(t_hbm, t_sh)

    pltpu.sync_copy(a_hbm, a_v)
    pltpu.sync_copy(b6_hbm, b6_v)
    plsc.subcore_barrier()

    ib = (i0, i1)
    pb = (pn0, pn1)
    rb = (r0, r1)
    tb = (t0, t1)
    isem = (si0, si1)
    tsem = (st0, st1)
    descs = {}

    def fire_idx(g):
        off = base + g * _C
        b = g % 2
        descs[("i", g)] = [
            pltpu.async_copy(item_hbm.at[pl.ds(off, _C)], ib[b], isem[b]),
            pltpu.async_copy(person_hbm.at[pl.ds(off, _C)], pb[b], isem[b]),
            pltpu.async_copy(resp_hbm.at[pl.ds(off, _C)], rb[b], isem[b]),
        ]

    def fire_t(g):
        b = g % 2
        descs[("t", g)] = pltpu.async_copy(t_sh.at[pb[b]], tb[b], tsem[b])

    lane = lax.iota(jnp.int32, _L)

    def compute(g, acc_in):
        b = g % 2
        off = base + g * _C

        @plsc.parallel_loop(0, _C, step=_L, unroll=4, carry=acc_in)
        def acc_out(i, acc):
            s = pl.ds(i, _L)
            it = ib[b][s]
            rs = rb[b][s]
            fhi = it * 6 + rs
            a16 = plsc.load_gather(a_v, [it])
            bl = plsc.load_gather(b6_v, [fhi - 1])
            bu = plsc.load_gather(b6_v, [fhi])
            tt = tb[b][s]
            zl = jnp.maximum(a16 * (tt - bl), -30.0)
            zu = jnp.maximum(a16 * (tt - bu), -30.0)
            x = jnp.exp(-zl)
            y = jnp.exp(-zu)
            p = (y - x) / ((1.0 + x) * (1.0 + y))
            lnp = _ln16(jnp.maximum(p, 1e-37))
            pos = off + i + lane
            return acc + jnp.where(pos < N_RESP, lnp, 0.0)

        return acc_out

    fire_idx(0)
    for d in descs[("i", 0)]:
        d.wait()
    fire_t(0)
    if _NCHUNK > 1:
        fire_idx(1)
    acc = jnp.zeros((_L,), jnp.float32)
    for g in range(_NCHUNK):
        descs[("t", g)].wait()
        if g + 1 < _NCHUNK:
            for d in descs[("i", g + 1)]:
                d.wait()
            fire_t(g + 1)
        acc = compute(g, acc)
        if g + 2 < _NCHUNK:
            fire_idx(g + 2)
    acc_v[...] = acc
    pltpu.async_copy(acc_v, out_hbm.at[wid], so).wait()


@functools.lru_cache(maxsize=1)
def _build_sc_gather():
    return pl.kernel(
        _sc_body,
        out_type=jax.ShapeDtypeStruct((_NW, _L), jnp.float32),
        mesh=plsc.VectorSubcoreMesh(
            core_axis_name="c", subcore_axis_name="s",
            num_cores=_NC, num_subcores=_NS),
        scratch_types=[
            pltpu.VMEM((N_ITEMS,), jnp.float32),
            pltpu.VMEM((6 * N_ITEMS,), jnp.float32),
            pltpu.VMEM_SHARED((N_PERSONS,), jnp.float32),
            pltpu.VMEM((_L,), jnp.float32),
            pltpu.VMEM((_C,), jnp.int32),
            pltpu.VMEM((_C,), jnp.int32),
            pltpu.VMEM((_C,), jnp.int32),
            pltpu.VMEM((_C,), jnp.int32),
            pltpu.VMEM((_C,), jnp.int32),
            pltpu.VMEM((_C,), jnp.int32),
            pltpu.VMEM((_C,), jnp.float32),
            pltpu.VMEM((_C,), jnp.float32),
            pltpu.SemaphoreType.DMA,
            pltpu.SemaphoreType.DMA,
            pltpu.SemaphoreType.DMA,
            pltpu.SemaphoreType.DMA,
            pltpu.SemaphoreType.DMA,
        ],
        compiler_params=pltpu.CompilerParams(needs_layout_passes=False),
    )


def _finish_body(ps_ref, a_ref, b6_ref, t_ref, bpm_ref, bps_ref, lvl_ref,
                 out_ref):
    ll = jnp.sum(ps_ref[...])

    a = a_ref[...]
    lp = jnp.sum(-0.5 * a * a) - 0.5 * _LOG2PI * N_ITEMS

    b = b6_ref[...][:, 1:5]
    lvl = lvl_ref[...]
    bpm = bpm_ref[...]
    bps = _softplus(bps_ref[...])
    mean = jnp.zeros((N_ITEMS, N_GRADES - 1), jnp.float32)
    std = jnp.zeros((N_ITEMS, N_GRADES - 1), jnp.float32)
    for l in range(N_LEVELS):
        m = (lvl == l).astype(jnp.float32)
        mean = mean + m * bpm[l:l + 1, :]
        std = std + m * bps[l:l + 1, :]
    z = (b - mean) / std
    lp += jnp.sum(-0.5 * z * z - jnp.log(std)) \
        - 0.5 * _LOG2PI * (N_ITEMS * (N_GRADES - 1))

    t = t_ref[...]
    lp += jnp.sum(-0.5 * t * t) - 0.5 * _LOG2PI * N_PERSONS
    lp += jnp.sum(-0.5 * bpm * bpm) - 0.5 * _LOG2PI * (N_LEVELS * (N_GRADES - 1))
    lp += jnp.sum(-2.0 * jnp.log(bps) - 1.0 / bps)

    out_ref[...] = jnp.reshape(-(ll + lp), (1, 1))


def _finish(psums, atab, b6, t2d, bpm, bps_, lvl2d):
    return pl.pallas_call(
        _finish_body,
        out_shape=jax.ShapeDtypeStruct((1, 1), jnp.float32),
    )(psums, atab, b6, t2d, bpm, bps_, lvl2d)


def kernel(a_, b_base_, b_diff_, t, b_prior_mean, b_prior_std_, indices,
           level_index):
    atab, b6 = _make_tables(a_.reshape(N_ITEMS, 1), b_base_, b_diff_)

    npad = _NPAD - N_RESP
    zpad = jnp.zeros((npad,), jnp.int32)
    item = jnp.concatenate([indices[:, 0].astype(jnp.int32), zpad])
    person = jnp.concatenate([indices[:, 1].astype(jnp.int32), zpad])
    resp = jnp.concatenate([indices[:, 2].astype(jnp.int32),
                            jnp.ones((npad,), jnp.int32)])

    psums = _build_sc_gather()(atab.reshape(N_ITEMS), b6.reshape(6 * N_ITEMS),
                               t, item, person, resp)

    out = _finish(psums, atab, b6,
                  t.reshape(800, 125), b_prior_mean, b_prior_std_,
                  level_index.reshape(N_ITEMS, 1).astype(jnp.int32))
    return out.reshape(())
